# baseline (speedup 1.0000x reference)
import jax
import jax.numpy as jnp
from jax.experimental import pallas as pl
from jax.experimental.pallas import tpu as pltpu
from jax.sharding import Mesh, NamedSharding, PartitionSpec as P

_N_EXPERTS = 64
_TOP_K = 2
_BLOCK = 2048


def _gate_kernel(x_ref, w_ref, idx_ref, wgt_ref):
    w = w_ref[...]
    logits = jax.lax.dot_general(
        x_ref[...], w, (((1,), (1,)), ((), ())),
        preferred_element_type=jnp.float32,
    )
    lane = jax.lax.broadcasted_iota(jnp.int32, logits.shape, 1)
    m1 = jnp.max(logits, axis=-1, keepdims=True)
    idx1 = jnp.min(jnp.where(logits == m1, lane, _N_EXPERTS),
                   axis=-1, keepdims=True)
    masked = jnp.where(lane == idx1, -jnp.inf, logits)
    m2 = jnp.max(masked, axis=-1, keepdims=True)
    idx2 = jnp.min(jnp.where(masked == m2, lane, _N_EXPERTS),
                   axis=-1, keepdims=True)
    e = jnp.exp(logits - m1)
    s = jnp.sum(e, axis=-1, keepdims=True)
    w1 = 1.0 / s
    w2 = jnp.exp(m2 - m1) / s
    idx_ref[...] = jnp.concatenate([idx1, idx2], axis=1)
    wgt_ref[...] = jnp.concatenate([w1, w2], axis=1)


def _gate_block(x, weight):
    n, h = x.shape
    return pl.pallas_call(
        _gate_kernel,
        grid=(n // _BLOCK,),
        in_specs=[
            pl.BlockSpec((_BLOCK, h), lambda i: (i, 0)),
            pl.BlockSpec((_N_EXPERTS, h), lambda i: (0, 0)),
        ],
        out_specs=[
            pl.BlockSpec((_BLOCK, _TOP_K), lambda i: (i, 0)),
            pl.BlockSpec((_BLOCK, _TOP_K), lambda i: (i, 0)),
        ],
        out_shape=[
            jax.ShapeDtypeStruct((n, _TOP_K), jnp.int32),
            jax.ShapeDtypeStruct((n, _TOP_K), jnp.float32),
        ],
        compiler_params=pltpu.CompilerParams(
            dimension_semantics=("arbitrary",),
        ),
    )(x, weight)


def kernel(hidden_states, weight):
    b, seq_len, h = hidden_states.shape
    n = b * seq_len
    x = hidden_states.reshape(n, h)
    devs = jax.devices()
    nd = len(devs)
    if nd > 1 and n % (nd * _BLOCK) == 0:
        mesh = Mesh(devs, ("d",))
        xs = jax.device_put(x, NamedSharding(mesh, P("d", None)))
        ws = jax.device_put(weight, NamedSharding(mesh, P()))
        idx, wgt = jax.shard_map(
            _gate_block, mesh=mesh,
            in_specs=(P("d", None), P()),
            out_specs=(P("d", None), P("d", None)),
            check_vma=False,
        )(xs, ws)
    else:
        idx, wgt = _gate_block(x, weight)
    return idx, wgt


# speedup vs baseline: 6.5331x; 6.5331x over previous
import functools

import jax
import jax.numpy as jnp
from jax import lax
from jax.experimental import pallas as pl
from jax.experimental.pallas import tpu as pltpu
from jax.experimental.pallas import tpu_sc as plsc

_N_EXPERTS = 64
_TOP_K = 2
_BLOCK = 2048


def _logits_t_kernel(x_ref, w_ref, o_ref):
    o_ref[...] = jax.lax.dot_general(
        w_ref[...], x_ref[...], (((1,), (1,)), ((), ())),
        preferred_element_type=jnp.float32,
    )


def _tc_logits_t(x, weight):
    n, h = x.shape
    return pl.pallas_call(
        _logits_t_kernel,
        grid=(n // _BLOCK,),
        in_specs=[
            pl.BlockSpec((_BLOCK, h), lambda i: (i, 0)),
            pl.BlockSpec((_N_EXPERTS, h), lambda i: (0, 0)),
        ],
        out_specs=pl.BlockSpec((_N_EXPERTS, _BLOCK), lambda i: (0, i)),
        out_shape=jax.ShapeDtypeStruct((_N_EXPERTS, n), jnp.float32),
    )(x, weight)


def _make_sc_gate(n_rows):
    info = plsc.get_sparse_core_info()
    nw = info.num_cores * info.num_subcores
    rows_w = n_rows // nw
    groups = rows_w // 16
    mesh = plsc.VectorSubcoreMesh(core_axis_name="c", subcore_axis_name="s")

    @functools.partial(
        pl.kernel, mesh=mesh,
        out_type=[
            jax.ShapeDtypeStruct((n_rows,), jnp.int32),
            jax.ShapeDtypeStruct((n_rows,), jnp.int32),
            jax.ShapeDtypeStruct((n_rows,), jnp.float32),
            jax.ShapeDtypeStruct((n_rows,), jnp.float32),
        ],
        scratch_types=[
            pltpu.VMEM((_N_EXPERTS, rows_w), jnp.float32),
            pltpu.VMEM((rows_w,), jnp.int32),
            pltpu.VMEM((rows_w,), jnp.int32),
            pltpu.VMEM((rows_w,), jnp.float32),
            pltpu.VMEM((rows_w,), jnp.float32),
        ],
    )
    def sc_gate(lt_hbm, i1_hbm, i2_hbm, w1_hbm, w2_hbm,
                ltbuf, i1buf, i2buf, w1buf, w2buf):
        wid = lax.axis_index("s") * info.num_cores + lax.axis_index("c")
        base = wid * rows_w
        pltpu.sync_copy(lt_hbm.at[:, pl.ds(base, rows_w)], ltbuf)
        neg_inf = jnp.full((16,), -jnp.inf, dtype=jnp.float32)
        zero_i = jnp.zeros((16,), dtype=jnp.int32)

        def per_group(g, c):
            toff = g * 16

            def scan_expert(e, carry):
                m1, i1, m2, i2 = carry
                v = ltbuf[e, pl.ds(toff, 16)]
                ev = jnp.broadcast_to(e, (16,))
                gt1 = v > m1
                gt2 = v > m2
                n_m2 = jnp.where(gt1, m1, jnp.where(gt2, v, m2))
                n_i2 = jnp.where(gt1, i1, jnp.where(gt2, ev, i2))
                n_m1 = jnp.where(gt1, v, m1)
                n_i1 = jnp.where(gt1, ev, i1)
                return n_m1, n_i1, n_m2, n_i2

            m1, i1, m2, i2 = lax.fori_loop(
                0, _N_EXPERTS, scan_expert,
                (neg_inf, zero_i, neg_inf, zero_i),
            )

            def sum_expert(e, s):
                return s + jnp.exp(ltbuf[e, pl.ds(toff, 16)] - m1)

            s = lax.fori_loop(
                0, _N_EXPERTS, sum_expert, jnp.zeros((16,), jnp.float32)
            )
            i1buf[pl.ds(toff, 16)] = i1
            i2buf[pl.ds(toff, 16)] = i2
            w1buf[pl.ds(toff, 16)] = 1.0 / s
            w2buf[pl.ds(toff, 16)] = jnp.exp(m2 - m1) / s
            return c

        lax.fori_loop(0, groups, per_group, 0)
        pltpu.sync_copy(i1buf, i1_hbm.at[pl.ds(base, rows_w)])
        pltpu.sync_copy(i2buf, i2_hbm.at[pl.ds(base, rows_w)])
        pltpu.sync_copy(w1buf, w1_hbm.at[pl.ds(base, rows_w)])
        pltpu.sync_copy(w2buf, w2_hbm.at[pl.ds(base, rows_w)])

    return sc_gate


def kernel(hidden_states, weight):
    b, seq_len, h = hidden_states.shape
    n = b * seq_len
    x = hidden_states.reshape(n, h)
    logits_t = _tc_logits_t(x, weight)
    i1, i2, w1, w2 = _make_sc_gate(n)(logits_t)
    return jnp.stack([i1, i2], axis=1), jnp.stack([w1, w2], axis=1)


# speedup vs baseline: 8.7827x; 1.3443x over previous
import functools

import jax
import jax.numpy as jnp
from jax.experimental import pallas as pl
from jax.experimental.pallas import tpu as pltpu

_N_EXPERTS = 64
_TOP_K = 2
_BLOCK = 2048


def _gate_kernel(xa_ref, xb_ref, xc_ref, xd_ref, w_ref, idx_ref, wgt_ref):
    w = w_ref[...]
    q = xa_ref.shape[1]
    parts = (xa_ref, xb_ref, xc_ref, xd_ref)
    logits = sum(
        jax.lax.dot_general(
            p[...], w[:, i * q:(i + 1) * q], (((1,), (1,)), ((), ())),
            preferred_element_type=jnp.float32,
        )
        for i, p in enumerate(parts)
    )
    lane = jax.lax.broadcasted_iota(jnp.int32, logits.shape, 1)
    m1 = jnp.max(logits, axis=-1, keepdims=True)
    idx1 = jnp.min(jnp.where(logits == m1, lane, _N_EXPERTS),
                   axis=-1, keepdims=True)
    masked = jnp.where(lane == idx1, -jnp.inf, logits)
    m2 = jnp.max(masked, axis=-1, keepdims=True)
    idx2 = jnp.min(jnp.where(masked == m2, lane, _N_EXPERTS),
                   axis=-1, keepdims=True)
    e = jnp.exp(logits - m1)
    s = jnp.sum(e, axis=-1, keepdims=True)
    w1 = 1.0 / s
    w2 = jnp.exp(m2 - m1) / s
    idx_ref[...] = jnp.concatenate([idx1, idx2], axis=1)
    wgt_ref[...] = jnp.concatenate([w1, w2], axis=1)


@functools.partial(jax.jit, static_argnames=())
def kernel(hidden_states, weight):
    b, seq_len, h = hidden_states.shape
    n = b * seq_len
    x = hidden_states.reshape(n, h)
    grid = (n // _BLOCK,)
    idx, wgt = pl.pallas_call(
        _gate_kernel,
        grid=grid,
        in_specs=[
            pl.BlockSpec((_BLOCK, h // 4), lambda i: (i, 0)),
            pl.BlockSpec((_BLOCK, h // 4), lambda i: (i, 1)),
            pl.BlockSpec((_BLOCK, h // 4), lambda i: (i, 2)),
            pl.BlockSpec((_BLOCK, h // 4), lambda i: (i, 3)),
            pl.BlockSpec((_N_EXPERTS, h), lambda i: (0, 0)),
        ],
        out_specs=[
            pl.BlockSpec((_BLOCK, _TOP_K), lambda i: (i, 0)),
            pl.BlockSpec((_BLOCK, _TOP_K), lambda i: (i, 0)),
        ],
        out_shape=[
            jax.ShapeDtypeStruct((n, _TOP_K), jnp.int32),
            jax.ShapeDtypeStruct((n, _TOP_K), jnp.float32),
        ],
        compiler_params=pltpu.CompilerParams(
            dimension_semantics=("parallel",),
        ),
    )(x, x, x, x, weight)
    return idx, wgt


# speedup vs baseline: 11.5567x; 1.3158x over previous
import functools

import jax
import jax.numpy as jnp
from jax.experimental import pallas as pl
from jax.experimental.pallas import tpu as pltpu

_N_EXPERTS = 64
_TOP_K = 2
_BLOCK = 2048


def _gate_kernel(xa_ref, xb_ref, xc_ref, xd_ref, w_ref, idx_ref, wgt_ref):
    w = w_ref[...]
    q = xa_ref.shape[1]
    parts = (xa_ref, xb_ref, xc_ref, xd_ref)
    logits = sum(
        jax.lax.dot_general(
            p[...], w[:, i * q:(i + 1) * q], (((1,), (1,)), ((), ())),
            preferred_element_type=jnp.float32,
        )
        for i, p in enumerate(parts)
    )
    lane = jax.lax.broadcasted_iota(jnp.int32, logits.shape, 1)
    m1 = jnp.max(logits, axis=-1, keepdims=True)
    idx1 = jnp.min(jnp.where(logits == m1, lane, _N_EXPERTS),
                   axis=-1, keepdims=True)
    masked = jnp.where(lane == idx1, -jnp.inf, logits)
    m2 = jnp.max(masked, axis=-1, keepdims=True)
    idx2 = jnp.min(jnp.where(masked == m2, lane, _N_EXPERTS),
                   axis=-1, keepdims=True)
    e = jnp.exp(logits - m1)
    s = jnp.sum(e, axis=-1, keepdims=True)
    w1 = 1.0 / s
    w2 = jnp.exp(m2 - m1) / s
    n_rows = idx1.shape[0]
    idx_ref[...] = jnp.concatenate(
        [idx1.reshape(1, n_rows), idx2.reshape(1, n_rows)], axis=0)
    wgt_ref[...] = jnp.concatenate(
        [w1.reshape(1, n_rows), w2.reshape(1, n_rows)], axis=0)


@functools.partial(jax.jit, static_argnames=())
def kernel(hidden_states, weight):
    b, seq_len, h = hidden_states.shape
    n = b * seq_len
    x = hidden_states.reshape(n, h)
    grid = (n // _BLOCK,)
    idx, wgt = pl.pallas_call(
        _gate_kernel,
        grid=grid,
        in_specs=[
            pl.BlockSpec((_BLOCK, h // 4), lambda i: (i, 0)),
            pl.BlockSpec((_BLOCK, h // 4), lambda i: (i, 1)),
            pl.BlockSpec((_BLOCK, h // 4), lambda i: (i, 2)),
            pl.BlockSpec((_BLOCK, h // 4), lambda i: (i, 3)),
            pl.BlockSpec((_N_EXPERTS, h), lambda i: (0, 0)),
        ],
        out_specs=[
            pl.BlockSpec((_TOP_K, _BLOCK), lambda i: (0, i)),
            pl.BlockSpec((_TOP_K, _BLOCK), lambda i: (0, i)),
        ],
        out_shape=[
            jax.ShapeDtypeStruct((_TOP_K, n), jnp.int32),
            jax.ShapeDtypeStruct((_TOP_K, n), jnp.float32),
        ],
        compiler_params=pltpu.CompilerParams(
            dimension_semantics=("parallel",),
        ),
    )(x, x, x, x, weight)
    return idx.T, wgt.T
